# Initial kernel scaffold; baseline (speedup 1.0000x reference)
import jax
import jax.numpy as jnp
from jax.experimental import pallas as pl


def kernel(pred, target):
    raise NotImplementedError("write your pallas kernel here")


# speedup vs baseline: 2.3920x; 2.3920x over previous
import functools

import jax
import jax.numpy as jnp
from jax import lax
from jax.experimental import pallas as pl
from jax.experimental.pallas import tpu as pltpu
from jax.experimental.pallas import tpu_sc as plsc

B = 64
C = 3
H = 512
W = 512
ROW = H * W
NBLK = 128
RPB = H // NBLK
BLK = ROW // NBLK
TOPK = 8
NWORKERS = 32
ROWS_PER_W = B // NWORKERS
_BIG = 1 << 20


def _tc_diff_body(pred_ref, target_ref, diff_ref, mx_ref):
    p = pred_ref[0]
    t = target_ref[0]
    d = t - p
    d = d * d
    s = d[0] + d[1] + d[2]
    diff_ref[0] = s
    s3 = s.reshape(NBLK, RPB, W)
    m = jnp.maximum(jnp.maximum(s3[:, 0, :], s3[:, 1, :]),
                    jnp.maximum(s3[:, 2, :], s3[:, 3, :]))
    mx_ref[0, 0] = jnp.max(m, axis=1)


def _tc_diff(pred, target):
    return pl.pallas_call(
        _tc_diff_body,
        grid=(B,),
        in_specs=[
            pl.BlockSpec((1, C, H, W), lambda b: (b, 0, 0, 0)),
            pl.BlockSpec((1, C, H, W), lambda b: (b, 0, 0, 0)),
        ],
        out_specs=[
            pl.BlockSpec((1, H, W), lambda b: (b, 0, 0)),
            pl.BlockSpec((1, 1, NBLK), lambda b: (b, 0, 0)),
        ],
        out_shape=[
            jax.ShapeDtypeStruct((B, H, W), jnp.float32),
            jax.ShapeDtypeStruct((B, 1, NBLK), jnp.float32),
        ],
    )(pred, target)


def _sc_row(row, mx_hbm, diff_hbm, out_hbm, mbuf, pool, obuf, sem):
    pltpu.sync_copy(mx_hbm.at[row], mbuf)
    nv = NBLK // 16
    vs = [mbuf[pl.ds(j * 16, 16)] for j in range(nv)]
    iotas = [lax.iota(jnp.int32, 16) + j * 16 for j in range(nv)]

    copies = []
    for it in range(TOPK):
        m = vs[0]
        for j in range(1, nv):
            m = jnp.maximum(m, vs[j])
        mx = jnp.max(m)
        cand = jnp.where(vs[0] == mx, iotas[0], _BIG)
        for j in range(1, nv):
            cand = jnp.minimum(cand, jnp.where(vs[j] == mx, iotas[j], _BIG))
        istar = jnp.min(cand)
        copies.append(pltpu.async_copy(
            diff_hbm.at[row, pl.ds(istar * BLK, BLK)],
            pool.at[pl.ds(it * BLK, BLK)], sem))
        for j in range(nv):
            vs[j] = jnp.where(iotas[j] == istar, jnp.float32(-1.0), vs[j])
    for cp in copies:
        cp.wait()

    unroll = 8
    nvec = TOPK * BLK // 16

    def body(i, carry):
        ms = list(carry)
        for u in range(unroll):
            v = pool[pl.ds((i * unroll + u) * 16, 16)]
            t = v
            for r in range(TOPK):
                hi = jnp.maximum(ms[r], t)
                t = jnp.minimum(ms[r], t)
                ms[r] = hi
        return tuple(ms)

    init = tuple(jnp.full((16,), -1.0, jnp.float32) for _ in range(TOPK))
    ms = list(lax.fori_loop(0, nvec // unroll, body, init))

    total = jnp.zeros((16,), jnp.float32)
    remaining = jnp.int32(TOPK)
    for _ in range(TOPK):
        m = ms[0]
        for r in range(1, TOPK):
            m = jnp.maximum(m, ms[r])
        mx = jnp.max(m)
        cnt = jnp.zeros((16,), jnp.int32)
        for r in range(TOPK):
            cnt = cnt + jnp.where(ms[r] == mx, jnp.int32(1), jnp.int32(0))
        c = jnp.sum(cnt)
        take = jnp.minimum(c, remaining)
        total = total + jnp.broadcast_to(mx * take.astype(jnp.float32), (16,))
        for r in range(TOPK):
            ms[r] = jnp.where(ms[r] == mx, jnp.float32(-1.0), ms[r])
        remaining = remaining - take

    obuf[...] = total
    pltpu.sync_copy(obuf, out_hbm.at[row])


def _sc_topk_body(mx_hbm, diff_hbm, out_hbm, mbuf, pool, obuf, sem):
    wid = lax.axis_index("s") * 2 + lax.axis_index("c")
    for rr in range(ROWS_PER_W):
        _sc_row(wid * ROWS_PER_W + rr,
                mx_hbm, diff_hbm, out_hbm, mbuf, pool, obuf, sem)


def _sc_topk(mx, diff2d):
    fn = pl.kernel(
        _sc_topk_body,
        out_type=jax.ShapeDtypeStruct((B, 16), jnp.float32),
        mesh=plsc.VectorSubcoreMesh(
            core_axis_name="c", subcore_axis_name="s",
            num_cores=2, num_subcores=16),
        scratch_types=[
            pltpu.VMEM((NBLK,), jnp.float32),
            pltpu.VMEM((TOPK * BLK,), jnp.float32),
            pltpu.VMEM((16,), jnp.float32),
            pltpu.SemaphoreType.DMA,
        ],
        compiler_params=pltpu.CompilerParams(needs_layout_passes=False),
    )
    return fn(mx, diff2d)


def _tc_mean_body(sums_ref, out_ref):
    s = jnp.sum(sums_ref[...][:, 0:1], keepdims=True)
    out_ref[...] = s / jnp.float32(B * TOPK)


def _tc_mean(sums):
    return pl.pallas_call(
        _tc_mean_body,
        out_shape=jax.ShapeDtypeStruct((1, 1), jnp.float32),
    )(sums)


def kernel(pred, target):
    diff, mx = _tc_diff(pred, target)
    sums = _sc_topk(mx.reshape(B, NBLK), diff.reshape(B, ROW))
    return _tc_mean(sums)[0, 0]


# speedup vs baseline: 3.2676x; 1.3661x over previous
import jax
import jax.numpy as jnp
from jax import lax
from jax.experimental import pallas as pl
from jax.experimental.pallas import tpu as pltpu
from jax.experimental.pallas import tpu_sc as plsc

B = 64
C = 3
H = 512
W = 512
ROW = H * W
NBLK = 128
RPB = H // NBLK
BLK = ROW // NBLK
TOPK = 8
NWORKERS = 32
ROWS_PER_W = B // NWORKERS
_BIG = 1 << 20


def _tc_max_body(pred_ref, target_ref, mx_ref):
    p = pred_ref[0]
    t = target_ref[0]
    d = t - p
    d = d * d
    s = d[0] + d[1] + d[2]
    s3 = s.reshape(NBLK, RPB, W)
    m = jnp.maximum(jnp.maximum(s3[:, 0, :], s3[:, 1, :]),
                    jnp.maximum(s3[:, 2, :], s3[:, 3, :]))
    mx_ref[0, 0] = jnp.max(m, axis=1)


def _tc_max(pred, target):
    return pl.pallas_call(
        _tc_max_body,
        grid=(B,),
        in_specs=[
            pl.BlockSpec((1, C, H, W), lambda b: (b, 0, 0, 0)),
            pl.BlockSpec((1, C, H, W), lambda b: (b, 0, 0, 0)),
        ],
        out_specs=pl.BlockSpec((1, 1, NBLK), lambda b: (b, 0, 0)),
        out_shape=jax.ShapeDtypeStruct((B, 1, NBLK), jnp.float32),
    )(pred, target)


def _sc_row(row, mx_hbm, pred_hbm, target_hbm, out_hbm,
            mbuf, pb, tb, obuf, sem):
    pltpu.sync_copy(mx_hbm.at[row, 0], mbuf)
    nv = NBLK // 16
    vs = [mbuf[pl.ds(j * 16, 16)] for j in range(nv)]
    iotas = [lax.iota(jnp.int32, 16) + j * 16 for j in range(nv)]

    copies = []
    for it in range(TOPK):
        m = vs[0]
        for j in range(1, nv):
            m = jnp.maximum(m, vs[j])
        mx = jnp.max(m)
        cand = jnp.where(vs[0] == mx, iotas[0], _BIG)
        for j in range(1, nv):
            cand = jnp.minimum(cand, jnp.where(vs[j] == mx, iotas[j], _BIG))
        istar = jnp.min(cand)
        r0 = istar * RPB
        for ch in range(C):
            copies.append(pltpu.async_copy(
                pred_hbm.at[row, ch, pl.ds(r0, RPB)], pb.at[it, ch], sem))
            copies.append(pltpu.async_copy(
                target_hbm.at[row, ch, pl.ds(r0, RPB)], tb.at[it, ch], sem))
        for j in range(nv):
            vs[j] = jnp.where(iotas[j] == istar, jnp.float32(-1.0), vs[j])
    for cp in copies:
        cp.wait()

    unroll = 8
    nvec = TOPK * BLK // 16

    def body(i, carry):
        ms = list(carry)
        for u in range(unroll):
            idx = i * unroll + u
            it = lax.shift_right_logical(idx, 7)
            v = idx - it * 128
            r = lax.shift_right_logical(v, 5)
            col = (v - r * 32) * 16
            acc = None
            for ch in range(C):
                x = (tb[it, ch, r, pl.ds(col, 16)]
                     - pb[it, ch, r, pl.ds(col, 16)])
                sq = x * x
                acc = sq if acc is None else acc + sq
            t = acc
            for k in range(TOPK):
                hi = jnp.maximum(ms[k], t)
                t = jnp.minimum(ms[k], t)
                ms[k] = hi
        return tuple(ms)

    init = tuple(jnp.full((16,), -1.0, jnp.float32) for _ in range(TOPK))
    ms = list(lax.fori_loop(0, nvec // unroll, body, init))

    total = jnp.zeros((16,), jnp.float32)
    remaining = jnp.int32(TOPK)
    for _ in range(TOPK):
        m = ms[0]
        for k in range(1, TOPK):
            m = jnp.maximum(m, ms[k])
        mx = jnp.max(m)
        cnt = jnp.zeros((16,), jnp.int32)
        for k in range(TOPK):
            cnt = cnt + jnp.where(ms[k] == mx, jnp.int32(1), jnp.int32(0))
        c = jnp.sum(cnt)
        take = jnp.minimum(c, remaining)
        total = total + jnp.broadcast_to(mx * take.astype(jnp.float32), (16,))
        for k in range(TOPK):
            ms[k] = jnp.where(ms[k] == mx, jnp.float32(-1.0), ms[k])
        remaining = remaining - take

    obuf[...] = total
    pltpu.sync_copy(obuf, out_hbm.at[row])


def _sc_topk_body(mx_hbm, pred_hbm, target_hbm, out_hbm,
                  mbuf, pb, tb, obuf, sem):
    wid = lax.axis_index("s") * 2 + lax.axis_index("c")

    def row_body(rr, carry):
        _sc_row(wid * ROWS_PER_W + rr, mx_hbm, pred_hbm, target_hbm,
                out_hbm, mbuf, pb, tb, obuf, sem)
        return carry

    lax.fori_loop(0, ROWS_PER_W, row_body, jnp.int32(0))


def _sc_topk(mx, pred, target):
    fn = pl.kernel(
        _sc_topk_body,
        out_type=jax.ShapeDtypeStruct((B, 16), jnp.float32),
        mesh=plsc.VectorSubcoreMesh(
            core_axis_name="c", subcore_axis_name="s",
            num_cores=2, num_subcores=16),
        scratch_types=[
            pltpu.VMEM((NBLK,), jnp.float32),
            pltpu.VMEM((TOPK, C, RPB, W), jnp.float32),
            pltpu.VMEM((TOPK, C, RPB, W), jnp.float32),
            pltpu.VMEM((16,), jnp.float32),
            pltpu.SemaphoreType.DMA,
        ],
        compiler_params=pltpu.CompilerParams(needs_layout_passes=False),
    )
    return fn(mx, pred, target)


def _tc_mean_body(sums_ref, out_ref):
    s = jnp.sum(sums_ref[...][:, 0:1], keepdims=True)
    out_ref[...] = s / jnp.float32(B * TOPK)


def _tc_mean(sums):
    return pl.pallas_call(
        _tc_mean_body,
        out_shape=jax.ShapeDtypeStruct((1, 1), jnp.float32),
    )(sums)


def kernel(pred, target):
    mx = _tc_max(pred, target)
    sums = _sc_topk(mx, pred, target)
    return _tc_mean(sums)[0, 0]


# speedup vs baseline: 3.6582x; 1.1195x over previous
import jax
import jax.numpy as jnp
from jax import lax
from jax.experimental import pallas as pl
from jax.experimental.pallas import tpu as pltpu
from jax.experimental.pallas import tpu_sc as plsc

B = 64
C = 3
H = 512
W = 512
ROW = H * W
NBLK = H
BLK = W
TOPK = 8
NWORKERS = 32
ROWS_PER_W = B // NWORKERS
_BIG = 1 << 20


def _tc_max_body(pred_ref, target_ref, mx_ref):
    p = pred_ref[0]
    t = target_ref[0]
    d = t - p
    d = d * d
    s = d[0] + d[1] + d[2]
    mx_ref[0, 0] = jnp.max(s, axis=1)


def _tc_max(pred, target):
    return pl.pallas_call(
        _tc_max_body,
        grid=(B,),
        in_specs=[
            pl.BlockSpec((1, C, H, W), lambda b: (b, 0, 0, 0)),
            pl.BlockSpec((1, C, H, W), lambda b: (b, 0, 0, 0)),
        ],
        out_specs=pl.BlockSpec((1, 1, NBLK), lambda b: (b, 0, 0)),
        out_shape=jax.ShapeDtypeStruct((B, 1, NBLK), jnp.float32),
    )(pred, target)


def _sc_row(row, mx_hbm, pred_hbm, target_hbm, out_hbm,
            mbuf, pb, tb, obuf, sem):
    pltpu.sync_copy(mx_hbm.at[row, 0], mbuf)
    nv = NBLK // 16
    iota = lax.iota(jnp.int32, 16)

    copies = []
    for it in range(TOPK):
        vs = [mbuf[pl.ds(j * 16, 16)] for j in range(nv)]
        m = vs[0]
        for j in range(1, nv):
            m = jnp.maximum(m, vs[j])
        mx = jnp.max(m)
        cand = jnp.where(vs[0] == mx, iota, _BIG)
        for j in range(1, nv):
            cand = jnp.minimum(cand, jnp.where(vs[j] == mx, iota + j * 16, _BIG))
        istar = jnp.min(cand)
        for ch in range(C):
            copies.append(pltpu.async_copy(
                pred_hbm.at[row, ch, pl.ds(istar, 1)], pb.at[it, ch], sem))
            copies.append(pltpu.async_copy(
                target_hbm.at[row, ch, pl.ds(istar, 1)], tb.at[it, ch], sem))
        g = lax.shift_right_logical(istar, 4)
        lane = istar - g * 16
        vg = mbuf[pl.ds(g * 16, 16)]
        mbuf[pl.ds(g * 16, 16)] = jnp.where(iota == lane,
                                            jnp.float32(-1.0), vg)
    for cp in copies:
        cp.wait()

    unroll = 8
    nvec = TOPK * BLK // 16

    def body(i, carry):
        ms = list(carry)
        for u in range(unroll):
            idx = i * unroll + u
            it = lax.shift_right_logical(idx, 5)
            col = (idx - it * 32) * 16
            acc = None
            for ch in range(C):
                x = (tb[it, ch, 0, pl.ds(col, 16)]
                     - pb[it, ch, 0, pl.ds(col, 16)])
                sq = x * x
                acc = sq if acc is None else acc + sq
            t = acc
            for k in range(TOPK):
                hi = jnp.maximum(ms[k], t)
                t = jnp.minimum(ms[k], t)
                ms[k] = hi
        return tuple(ms)

    init = tuple(jnp.full((16,), -1.0, jnp.float32) for _ in range(TOPK))
    ms = list(lax.fori_loop(0, nvec // unroll, body, init))

    total = jnp.zeros((16,), jnp.float32)
    remaining = jnp.int32(TOPK)
    for _ in range(TOPK):
        m = ms[0]
        for k in range(1, TOPK):
            m = jnp.maximum(m, ms[k])
        mx = jnp.max(m)
        cnt = jnp.zeros((16,), jnp.int32)
        for k in range(TOPK):
            cnt = cnt + jnp.where(ms[k] == mx, jnp.int32(1), jnp.int32(0))
        c = jnp.sum(cnt)
        take = jnp.minimum(c, remaining)
        total = total + jnp.broadcast_to(mx * take.astype(jnp.float32), (16,))
        for k in range(TOPK):
            ms[k] = jnp.where(ms[k] == mx, jnp.float32(-1.0), ms[k])
        remaining = remaining - take

    obuf[...] = total
    pltpu.sync_copy(obuf, out_hbm.at[row])


def _sc_topk_body(mx_hbm, pred_hbm, target_hbm, out_hbm,
                  mbuf, pb, tb, obuf, sem):
    wid = lax.axis_index("s") * 2 + lax.axis_index("c")

    def row_body(rr, carry):
        _sc_row(wid * ROWS_PER_W + rr, mx_hbm, pred_hbm, target_hbm,
                out_hbm, mbuf, pb, tb, obuf, sem)
        return carry

    lax.fori_loop(0, ROWS_PER_W, row_body, jnp.int32(0))


def _sc_topk(mx, pred, target):
    fn = pl.kernel(
        _sc_topk_body,
        out_type=jax.ShapeDtypeStruct((B, 16), jnp.float32),
        mesh=plsc.VectorSubcoreMesh(
            core_axis_name="c", subcore_axis_name="s",
            num_cores=2, num_subcores=16),
        scratch_types=[
            pltpu.VMEM((NBLK,), jnp.float32),
            pltpu.VMEM((TOPK, C, 1, BLK), jnp.float32),
            pltpu.VMEM((TOPK, C, 1, BLK), jnp.float32),
            pltpu.VMEM((16,), jnp.float32),
            pltpu.SemaphoreType.DMA,
        ],
        compiler_params=pltpu.CompilerParams(needs_layout_passes=False),
    )
    return fn(mx, pred, target)


def _tc_mean_body(sums_ref, out_ref):
    s = jnp.sum(sums_ref[...][:, 0:1], keepdims=True)
    out_ref[...] = s / jnp.float32(B * TOPK)


def _tc_mean(sums):
    return pl.pallas_call(
        _tc_mean_body,
        out_shape=jax.ShapeDtypeStruct((1, 1), jnp.float32),
    )(sums)


def kernel(pred, target):
    mx = _tc_max(pred, target)
    sums = _sc_topk(mx, pred, target)
    return _tc_mean(sums)[0, 0]
